# baseline (speedup 1.0000x reference)
import functools

import jax
import jax.numpy as jnp
from jax import lax
from jax.experimental import pallas as pl
from jax.experimental.pallas import tpu as pltpu
from jax.experimental.pallas import tpu_sc as plsc

_EMB = 32
_NSH = 4
_NW = 32
_L = 16
_C2 = 256
_G = 64


def _build(b_total, nlines):
    c = b_total // _NW
    nsub = c // _C2
    ngrp = _C2 // _L
    nk = _C2 // _G

    mesh = plsc.VectorSubcoreMesh(core_axis_name="c", subcore_axis_name="s")

    @functools.partial(
        pl.kernel,
        mesh=mesh,
        out_type=jax.ShapeDtypeStruct((b_total * _EMB,), jnp.float32),
        scratch_types=[
            pltpu.VMEM((c,), jnp.int32),
            pltpu.VMEM((_C2,), jnp.int32),
            pltpu.VMEM((_C2,), jnp.int32),
            pltpu.VMEM((_C2, 128), jnp.float32),
            pltpu.VMEM((_C2, 128), jnp.float32),
            pltpu.VMEM((_C2 * _EMB,), jnp.float32),
            pltpu.SemaphoreType.DMA,
            pltpu.SemaphoreType.DMA,
        ],
        compiler_params=pltpu.CompilerParams(needs_layout_passes=False),
    )
    def lookup(ids_h, tbl, out_h,
               ids_v, ll0, ll1, gbuf0, gbuf1, obuf, gs0, gs1):
        llA = (ll0, ll1)
        gbufA = (gbuf0, gbuf1)
        gsA = (gs0, gs1)

        wid = lax.axis_index("s") * 2 + lax.axis_index("c")
        wbase = wid * c
        pltpu.sync_copy(ids_h.at[pl.ds(wbase, c)], ids_v)

        def step(s, par):
            sbase = jnp.minimum(s, nsub - 1) * _C2

            def lgroup(g, _):
                v = ids_v[pl.ds(sbase + g * _L, _L)]
                line = (v & (_NSH - 1)) * nlines + lax.shift_right_logical(v, 4)
                llA[par][pl.ds(g * _L, _L)] = line
                return 0

            lax.fori_loop(0, ngrp, lgroup, 0)

            @pl.when(s < nsub)
            def _():
                for k in range(nk):
                    pltpu.make_async_copy(
                        tbl.at[llA[par].at[pl.ds(k * _G, _G)]],
                        gbufA[par].at[pl.ds(k * _G, _G)],
                        gsA[par],
                    ).start()

            @pl.when(s >= 1)
            def _():
                for k in range(nk):
                    pltpu.make_async_copy(
                        tbl.at[llA[1 - par].at[pl.ds(k * _G, _G)]],
                        gbufA[1 - par].at[pl.ds(k * _G, _G)],
                        gsA[1 - par],
                    ).wait()

            spbase = jnp.maximum(s - 1, 0) * _C2

            def egroup(g, _):
                v = ids_v[pl.ds(spbase + g * _L, _L)]
                col0 = (lax.shift_right_logical(v, 2) & (_NSH - 1)) * _EMB
                for l in range(_L):
                    cb = col0[l]
                    e = g * _L + l
                    obuf[pl.ds(e * _EMB, _L)] = \
                        gbufA[1 - par][e, pl.ds(cb, _L)]
                    obuf[pl.ds(e * _EMB + _L, _L)] = \
                        gbufA[1 - par][e, pl.ds(cb + _L, _L)]
                return 0

            lax.fori_loop(0, ngrp, egroup, 0)

            @pl.when(s >= 1)
            def _():
                pltpu.sync_copy(
                    obuf,
                    out_h.at[pl.ds((wbase + spbase) * _EMB, _C2 * _EMB)])

        def dbody(i, carry):
            step(2 * i, 0)
            step(2 * i + 1, 1)
            return carry

        lax.fori_loop(0, (nsub + 2) // 2, dbody, 0)

    return lookup


def kernel(inputs, emb_0, emb_1, emb_2, emb_3):
    batch, steps = inputs.shape
    b_total = batch * steps
    ids = inputs.reshape(b_total)
    nlines = emb_0.shape[0] // _NSH
    tbl = jnp.concatenate(
        [e.reshape(nlines, _NSH * _EMB) for e in (emb_0, emb_1, emb_2, emb_3)],
        axis=0)
    out = _build(b_total, nlines)(ids, tbl)
    return out.reshape(batch, steps, _EMB)
